# baseline (speedup 1.0000x reference)
import functools

import jax
import jax.numpy as jnp
from jax import lax
from jax.experimental import pallas as pl
from jax.experimental.pallas import tpu as pltpu
from jax.experimental.pallas import tpu_sc as plsc

B = 16384
FC = 13
NF = 26
NE = 32
NOUT = FC + NF
VOCAB = 100000
ROWW = VOCAB * NF
TBL = ROWW * NE
NC = 2
NS = 16
NW = NC * NS
NCHUNK = B // 128
CAT_UNITS = NF * NCHUNK // NW
CONT_UNITS = FC * NCHUNK // NW

_mesh = plsc.VectorSubcoreMesh(
    core_axis_name="c", subcore_axis_name="s", num_cores=NC, num_subcores=NS
)


@functools.partial(
    pl.kernel,
    out_type=jax.ShapeDtypeStruct((NOUT, 4, NCHUNK, 1024), jnp.float32),
    mesh=_mesh,
    compiler_params=pltpu.CompilerParams(use_tc_tiling_on_sc=False),
    scratch_types=[
        pltpu.VMEM((128,), jnp.int32),
        pltpu.VMEM((4096,), jnp.int32),
        pltpu.VMEM((4096,), jnp.float32),
        pltpu.VMEM((128,), jnp.float32),
        pltpu.VMEM((4096,), jnp.float32),
        pltpu.VMEM((NF, NE), jnp.int32),
        pltpu.VMEM((NF, NE), jnp.float32),
        pltpu.VMEM((FC, NE), jnp.float32),
        pltpu.VMEM((FC, NE), jnp.float32),
        pltpu.VMEM((FC, 16), jnp.float32),
        pltpu.SemaphoreType.DMA,
    ],
)
def _sc_embed(xt_hbm, xcat_t_hbm, wnanr_hbm, bases_hbm, wlin_hbm, bcat_hbm,
              wcont_hbm, bcont_hbm, out_hbm,
              codes, idxb, stage, xrow, cstage, basev, biasv, wcv, bcv,
              wnanv, sem):
    cax = lax.axis_index("c")
    sax = lax.axis_index("s")
    wid = sax * NC + cax

    pltpu.sync_copy(bcat_hbm, biasv)
    pltpu.sync_copy(bases_hbm, basev)
    pltpu.sync_copy(wcont_hbm, wcv)
    pltpu.sync_copy(bcont_hbm, bcv)
    pltpu.sync_copy(wnanr_hbm, wnanv)

    def cont_unit(t, _):
        u = wid + NW * t
        fo = u // NCHUNK
        cc = u % NCHUNK
        pltpu.sync_copy(xt_hbm.at[fo, pl.ds(cc * 128, 128)], xrow)
        wnrow = wnanv[fo, pl.ds(0, 16)]
        wv0 = wcv[fo, pl.ds(0, 16)]
        wv1 = wcv[fo, pl.ds(16, 16)]
        bv0 = bcv[fo, pl.ds(0, 16)]
        bv1 = bcv[fo, pl.ds(16, 16)]
        xq = []
        for i in range(8):
            v = xrow[pl.ds(i * 16, 16)]
            xq.append(jnp.where(v != v, wnrow, v))
        for e in range(NE):
            wsp = jnp.full((16,), (wv0 if e < 16 else wv1)[e % 16],
                           dtype=jnp.float32)
            bsp = jnp.full((16,), (bv0 if e < 16 else bv1)[e % 16],
                           dtype=jnp.float32)
            for i in range(8):
                cstage[pl.ds(e * 128 + i * 16, 16)] = xq[i] * wsp + bsp
        for r in range(4):
            pltpu.sync_copy(cstage.at[pl.ds(r * 1024, 1024)],
                            out_hbm.at[fo, r, cc])
        return 0

    lax.fori_loop(0, CONT_UNITS, cont_unit, 0)

    def cat_unit(t, _):
        u = wid + NW * t
        f = u // NCHUNK
        cc = u % NCHUNK
        pltpu.sync_copy(xcat_t_hbm.at[f, pl.ds(cc * 128, 128)], codes)
        ba0 = basev[f, pl.ds(0, 16)]
        ba1 = basev[f, pl.ds(16, 16)]
        cq = [codes[pl.ds(i * 16, 16)] for i in range(8)]
        for e in range(NE):
            bsp = jnp.full((16,), (ba0 if e < 16 else ba1)[e % 16],
                           dtype=jnp.int32)
            for i in range(8):
                idxb[pl.ds(e * 128 + i * 16, 16)] = cq[i] + bsp
        pltpu.async_copy(wlin_hbm.at[idxb], stage, sem).wait()
        bb0 = biasv[f, pl.ds(0, 16)]
        bb1 = biasv[f, pl.ds(16, 16)]
        for e in range(NE):
            bsp = jnp.full((16,), (bb0 if e < 16 else bb1)[e % 16],
                           dtype=jnp.float32)
            for i in range(8):
                stage[pl.ds(e * 128 + i * 16, 16)] += bsp
        for r in range(4):
            pltpu.sync_copy(stage.at[pl.ds(r * 1024, 1024)],
                            out_hbm.at[FC + f, r, cc])
        return 0

    lax.fori_loop(0, CAT_UNITS, cat_unit, 0)


def kernel(x_continuous, x_categorical, w_nan, offsets, w_categorical,
           b_categorical, w_continuous, b_continuous):
    xt = x_continuous.T
    xcat_t = x_categorical.T
    wlin = w_categorical.T.reshape(TBL)
    wnanr = jnp.broadcast_to(w_nan[:, None], (FC, 16))
    bases = (offsets[:, None]
             + jnp.arange(NE, dtype=jnp.int32)[None, :] * ROWW)
    out6 = _sc_embed(xt, xcat_t, wnanr, bases, wlin, b_categorical,
                     w_continuous, b_continuous)
    out = (out6.reshape(NOUT, 4, NCHUNK, 8, 128)
           .transpose(2, 4, 0, 1, 3)
           .reshape(B, NOUT, NE))
    return out


# speedup vs baseline: 3.8794x; 3.8794x over previous
import functools

import jax
import jax.numpy as jnp
from jax import lax
from jax.experimental import pallas as pl
from jax.experimental.pallas import tpu as pltpu
from jax.experimental.pallas import tpu_sc as plsc

B = 16384
FC = 13
NF = 26
NE = 32
NOUT = FC + NF
VOCAB = 100000
NC = 2
NS = 16
NW = NC * NS
NCHUNK = B // 128
NBLK = B // 1024
CAT_UNITS = NF * NBLK // NW
CONT_UNITS = FC * NCHUNK // NW

_mesh = plsc.VectorSubcoreMesh(
    core_axis_name="c", subcore_axis_name="s", num_cores=NC, num_subcores=NS
)


@functools.partial(
    pl.kernel,
    out_type=jax.ShapeDtypeStruct((NOUT, 4, NCHUNK, 8, 1, 128), jnp.float32),
    mesh=_mesh,
    compiler_params=pltpu.CompilerParams(use_tc_tiling_on_sc=False,
                                         needs_layout_passes=False),
    scratch_types=[
        pltpu.VMEM((8, 128), jnp.int32),
        pltpu.VMEM((1024, NE), jnp.float32),
        pltpu.VMEM((NE, 8, 1, 1, 128), jnp.float32),
        pltpu.VMEM((128,), jnp.float32),
        pltpu.VMEM((NE, 1, 1, 1, 128), jnp.float32),
        pltpu.VMEM((NF, 16), jnp.int32),
        pltpu.VMEM((NF, NE), jnp.float32),
        pltpu.VMEM((FC, NE), jnp.float32),
        pltpu.VMEM((FC, NE), jnp.float32),
        pltpu.VMEM((FC, 16), jnp.float32),
        pltpu.SemaphoreType.DMA,
    ],
)
def _sc_embed(xt_hbm, xcat_t_hbm, wnanr_hbm, offsr_hbm, wcat_hbm, bcat_hbm,
              wcont_hbm, bcont_hbm, out_hbm,
              idx2, rowbuf, estage, xrow, cstage, offsv, biasv, wcv, bcv,
              wnanv, sem):
    cax = lax.axis_index("c")
    sax = lax.axis_index("s")
    wid = sax * NC + cax

    pltpu.sync_copy(bcat_hbm, biasv)
    pltpu.sync_copy(offsr_hbm, offsv)
    pltpu.sync_copy(wcont_hbm, wcv)
    pltpu.sync_copy(bcont_hbm, bcv)
    pltpu.sync_copy(wnanr_hbm, wnanv)

    def cont_unit(t, _):
        u = wid + NW * t
        fo = u // NCHUNK
        cc = u % NCHUNK
        pltpu.sync_copy(xt_hbm.at[fo, pl.ds(cc * 128, 128)], xrow)
        wnrow = wnanv[fo, pl.ds(0, 16)]
        wv0 = wcv[fo, pl.ds(0, 16)]
        wv1 = wcv[fo, pl.ds(16, 16)]
        bv0 = bcv[fo, pl.ds(0, 16)]
        bv1 = bcv[fo, pl.ds(16, 16)]
        xq = []
        for i in range(8):
            v = xrow[pl.ds(i * 16, 16)]
            xq.append(jnp.where(v != v, wnrow, v))
        for e in range(NE):
            wsp = jnp.full((16,), (wv0 if e < 16 else wv1)[e % 16],
                           dtype=jnp.float32)
            bsp = jnp.full((16,), (bv0 if e < 16 else bv1)[e % 16],
                           dtype=jnp.float32)
            for i in range(8):
                cstage[e, 0, 0, 0, pl.ds(i * 16, 16)] = xq[i] * wsp + bsp
        for e in range(NE):
            pltpu.sync_copy(cstage.at[e],
                            out_hbm.at[fo, e // 8, pl.ds(cc, 1),
                                       pl.ds(e % 8, 1)])
        return 0

    lax.fori_loop(0, CONT_UNITS, cont_unit, 0)

    rowids = lax.iota(jnp.int32, 16)

    def cat_unit(t, _):
        u = wid + NW * t
        f = u // NBLK
        cb = u % NBLK
        for j in range(8):
            pltpu.sync_copy(xcat_t_hbm.at[f, pl.ds(cb * 1024 + j * 128, 128)],
                            idx2.at[j])
        osp = jnp.full((16,), offsv[f, pl.ds(0, 16)][0], dtype=jnp.int32)

        def addoff(j, _, osp=osp):
            for i in range(8):
                idx2[j, pl.ds(i * 16, 16)] += osp
            return 0

        lax.fori_loop(0, 8, addoff, 0)
        copies = [
            pltpu.async_copy(wcat_hbm.at[idx2.at[j]],
                             rowbuf.at[pl.ds(j * 128, 128)], sem)
            for j in range(8)
        ]
        for cp in copies:
            cp.wait()
        bb0 = biasv[f, pl.ds(0, 16)]
        bb1 = biasv[f, pl.ds(16, 16)]
        bsps = [jnp.full((16,), (bb0 if e < 16 else bb1)[e % 16],
                         dtype=jnp.float32) for e in range(NE)]

        def tbody(i, _):
            rid = i * 16 + rowids
            j = i // 8
            l16 = (i % 8) * 16
            for e in range(NE):
                v = plsc.load_gather(rowbuf, [rid, jnp.full((16,), e,
                                                            jnp.int32)])
                estage[e, j, 0, 0, pl.ds(l16, 16)] = v + bsps[e]
            return 0

        lax.fori_loop(0, 64, tbody, 0)
        for e in range(NE):
            pltpu.sync_copy(estage.at[e],
                            out_hbm.at[FC + f, e // 8, pl.ds(cb * 8, 8),
                                       pl.ds(e % 8, 1)])
        return 0

    lax.fori_loop(0, CAT_UNITS, cat_unit, 0)


def kernel(x_continuous, x_categorical, w_nan, offsets, w_categorical,
           b_categorical, w_continuous, b_continuous):
    xt = x_continuous.T
    xcat_t = x_categorical.T
    wnanr = jnp.broadcast_to(w_nan[:, None], (FC, 16))
    offsr = jnp.broadcast_to(offsets[:, None], (NF, 16))
    out6 = _sc_embed(xt, xcat_t, wnanr, offsr, w_categorical, b_categorical,
                     w_continuous, b_continuous)
    out = (out6.reshape(NOUT, 4, NCHUNK, 8, 128)
           .transpose(2, 4, 0, 1, 3)
           .reshape(B, NOUT, NE))
    return out


# speedup vs baseline: 3.9051x; 1.0066x over previous
import functools

import jax
import jax.numpy as jnp
from jax import lax
from jax.experimental import pallas as pl
from jax.experimental.pallas import tpu as pltpu
from jax.experimental.pallas import tpu_sc as plsc

B = 16384
FC = 13
NF = 26
NE = 32
NOUT = FC + NF
VOCAB = 100000
NC = 2
NS = 16
NW = NC * NS
NCHUNK = B // 128
NBLK = B // 1024
CAT_UNITS = NF * NBLK // NW
CONT_UNITS = FC * NCHUNK // NW

_mesh = plsc.VectorSubcoreMesh(
    core_axis_name="c", subcore_axis_name="s", num_cores=NC, num_subcores=NS
)


@functools.partial(
    pl.kernel,
    out_type=jax.ShapeDtypeStruct((NOUT, 4, NCHUNK, 8, 1, 128), jnp.float32),
    mesh=_mesh,
    compiler_params=pltpu.CompilerParams(use_tc_tiling_on_sc=False,
                                         needs_layout_passes=False),
    scratch_types=[
        pltpu.VMEM((8, 128), jnp.int32),
        pltpu.VMEM((1024, NE), jnp.float32),
        pltpu.VMEM((NE, 8, 1, 1, 128), jnp.float32),
        pltpu.VMEM((128,), jnp.float32),
        pltpu.VMEM((NE, 1, 1, 1, 128), jnp.float32),
        pltpu.VMEM((NF, 16), jnp.int32),
        pltpu.VMEM((NF, NE), jnp.float32),
        pltpu.VMEM((FC, NE), jnp.float32),
        pltpu.VMEM((FC, NE), jnp.float32),
        pltpu.VMEM((FC, 16), jnp.float32),
        pltpu.SemaphoreType.DMA,
    ],
)
def _sc_embed(xt_hbm, xcat_t_hbm, wnanr_hbm, offsr_hbm, wcat_hbm, bcat_hbm,
              wcont_hbm, bcont_hbm, out_hbm,
              idx2, rowbuf, estage, xrow, cstage, offsv, biasv, wcv, bcv,
              wnanv, sem):
    cax = lax.axis_index("c")
    sax = lax.axis_index("s")
    wid = sax * NC + cax

    pltpu.sync_copy(bcat_hbm, biasv)
    pltpu.sync_copy(offsr_hbm, offsv)
    pltpu.sync_copy(wcont_hbm, wcv)
    pltpu.sync_copy(bcont_hbm, bcv)
    pltpu.sync_copy(wnanr_hbm, wnanv)

    def cont_unit(t, _):
        u = wid + NW * t
        fo = u // NCHUNK
        cc = u % NCHUNK
        pltpu.sync_copy(xt_hbm.at[fo, pl.ds(cc * 128, 128)], xrow)
        wnrow = wnanv[fo, pl.ds(0, 16)]
        wv0 = wcv[fo, pl.ds(0, 16)]
        wv1 = wcv[fo, pl.ds(16, 16)]
        bv0 = bcv[fo, pl.ds(0, 16)]
        bv1 = bcv[fo, pl.ds(16, 16)]
        xq = []
        for i in range(8):
            v = xrow[pl.ds(i * 16, 16)]
            xq.append(jnp.where(v != v, wnrow, v))
        for e in range(NE):
            wsp = jnp.full((16,), (wv0 if e < 16 else wv1)[e % 16],
                           dtype=jnp.float32)
            bsp = jnp.full((16,), (bv0 if e < 16 else bv1)[e % 16],
                           dtype=jnp.float32)
            for i in range(8):
                cstage[e, 0, 0, 0, pl.ds(i * 16, 16)] = xq[i] * wsp + bsp
        for e in range(NE):
            pltpu.sync_copy(cstage.at[e],
                            out_hbm.at[fo, e // 8, pl.ds(cc, 1),
                                       pl.ds(e % 8, 1)])
        return 0

    lax.fori_loop(0, CONT_UNITS, cont_unit, 0)

    rowids = lax.iota(jnp.int32, 16)

    def cat_unit(t, _):
        u = wid + NW * t
        f = u // NBLK
        cb = u % NBLK
        for j in range(8):
            pltpu.sync_copy(xcat_t_hbm.at[f, pl.ds(cb * 1024 + j * 128, 128)],
                            idx2.at[j])
        osp = jnp.full((16,), offsv[f, pl.ds(0, 16)][0], dtype=jnp.int32)

        def addoff(j, _, osp=osp):
            for i in range(8):
                idx2[j, pl.ds(i * 16, 16)] += osp
            return 0

        lax.fori_loop(0, 8, addoff, 0)
        copies = [
            pltpu.async_copy(wcat_hbm.at[idx2.at[j]],
                             rowbuf.at[pl.ds(j * 128, 128)], sem)
            for j in range(8)
        ]
        for cp in copies:
            cp.wait()
        bb0 = biasv[f, pl.ds(0, 16)]
        bb1 = biasv[f, pl.ds(16, 16)]

        def baddr(r_, _, bb0=bb0, bb1=bb1):
            rowbuf[r_, pl.ds(0, 16)] += bb0
            rowbuf[r_, pl.ds(16, 16)] += bb1
            return 0

        lax.fori_loop(0, 1024, baddr, 0, unroll=4)

        def tbody(e, _):
            ecol = jnp.full((16,), e, dtype=jnp.int32)
            for i in range(64):
                v = plsc.load_gather(rowbuf, [i * 16 + rowids, ecol])
                estage[e, i // 8, 0, 0, pl.ds((i % 8) * 16, 16)] = v
            return 0

        lax.fori_loop(0, NE, tbody, 0)
        for e in range(NE):
            pltpu.sync_copy(estage.at[e],
                            out_hbm.at[FC + f, e // 8, pl.ds(cb * 8, 8),
                                       pl.ds(e % 8, 1)])
        return 0

    lax.fori_loop(0, CAT_UNITS, cat_unit, 0)


def kernel(x_continuous, x_categorical, w_nan, offsets, w_categorical,
           b_categorical, w_continuous, b_continuous):
    xt = x_continuous.T
    xcat_t = x_categorical.T
    wnanr = jnp.broadcast_to(w_nan[:, None], (FC, 16))
    offsr = jnp.broadcast_to(offsets[:, None], (NF, 16))
    out6 = _sc_embed(xt, xcat_t, wnanr, offsr, w_categorical, b_categorical,
                     w_continuous, b_continuous)
    out = (out6.reshape(NOUT, 4, NCHUNK, 8, 128)
           .transpose(2, 4, 0, 1, 3)
           .reshape(B, NOUT, NE))
    return out


# speedup vs baseline: 4.0979x; 1.0494x over previous
import functools

import jax
import jax.numpy as jnp
from jax import lax
from jax.experimental import pallas as pl
from jax.experimental.pallas import tpu as pltpu
from jax.experimental.pallas import tpu_sc as plsc

B = 16384
FC = 13
NF = 26
NE = 32
NOUT = FC + NF
NC = 2
NS = 16
NW = NC * NS
BLK = 1024
SUB = 128
NSUB = BLK // SUB
CB = 128
BPW = B // NW

_mesh = plsc.VectorSubcoreMesh(
    core_axis_name="c", subcore_axis_name="s", num_cores=NC, num_subcores=NS
)


@functools.partial(
    pl.kernel,
    out_type=jax.ShapeDtypeStruct((B, NOUT, NE), jnp.float32),
    mesh=_mesh,
    compiler_params=pltpu.CompilerParams(use_tc_tiling_on_sc=False),
    scratch_types=[
        pltpu.VMEM((NSUB, SUB), jnp.int32),
        pltpu.VMEM((BLK, NE), jnp.float32),
        pltpu.VMEM((NF, NE), jnp.float32),
        pltpu.VMEM((NF, 16), jnp.int32),
        pltpu.VMEM((FC, NE), jnp.float32),
        pltpu.VMEM((FC, NE), jnp.float32),
        pltpu.VMEM((16,), jnp.float32),
        pltpu.VMEM((CB, 16), jnp.float32),
        pltpu.VMEM((CB, FC, NE), jnp.float32),
        pltpu.SemaphoreType.DMA,
    ],
)
def _sc_embed(xc_hbm, xcat_t_hbm, wnan_hbm, offs_hbm, wcat_hbm, bcat_hbm,
              wcont_hbm, bcont_hbm, out_hbm,
              idx2, rows, biasv, offsv, wcv, bcv, wnanv, xcv, cstage, sem):
    c = lax.axis_index("c")
    s = lax.axis_index("s")
    wid = s * NC + c

    pltpu.sync_copy(bcat_hbm, biasv)
    pltpu.sync_copy(offs_hbm, offsv)
    pltpu.sync_copy(wcont_hbm, wcv)
    pltpu.sync_copy(bcont_hbm, bcv)
    pltpu.sync_copy(wnan_hbm, wnanv)

    wnv = wnanv[pl.ds(0, 16)]
    for cb in range(BPW // CB):
        b0 = wid * BPW + cb * CB
        pltpu.sync_copy(xc_hbm.at[pl.ds(b0, CB)], xcv)

        def cbody(r, _, wnv=wnv):
            xrow = xcv[r, pl.ds(0, 16)]
            for f in range(FC):
                sv = xrow[f]
                sv = jnp.where(sv != sv, wnv[f], sv)
                xb = jnp.full((16,), sv, dtype=jnp.float32)
                wv0 = wcv[f, pl.ds(0, 16)]
                wv1 = wcv[f, pl.ds(16, 16)]
                bv0 = bcv[f, pl.ds(0, 16)]
                bv1 = bcv[f, pl.ds(16, 16)]
                cstage[r, f, pl.ds(0, 16)] = xb * wv0 + bv0
                cstage[r, f, pl.ds(16, 16)] = xb * wv1 + bv1
            return 0

        lax.fori_loop(0, CB, cbody, 0, unroll=2)
        pltpu.sync_copy(cstage, out_hbm.at[pl.ds(b0, CB), pl.ds(0, FC)])

    blk = wid % 16
    grp = wid // 16
    b0 = blk * BLK
    for k in range(FC):
        f = grp * FC + k
        off_vec = offsv[f, pl.ds(0, 16)]

        for j in range(NSUB):
            pltpu.sync_copy(xcat_t_hbm.at[f, pl.ds(b0 + j * SUB, SUB)],
                            idx2.at[j])

        def obody(j, _, off_vec=off_vec):
            for i in range(SUB // 16):
                idx2[j, pl.ds(i * 16, 16)] += off_vec
            return 0

        lax.fori_loop(0, NSUB, obody, 0)

        copies = [
            pltpu.async_copy(wcat_hbm.at[idx2.at[j]],
                             rows.at[pl.ds(j * SUB, SUB)], sem)
            for j in range(NSUB)
        ]
        for cp in copies:
            cp.wait()

        bv0 = biasv[f, pl.ds(0, 16)]
        bv1 = biasv[f, pl.ds(16, 16)]

        def bbody(r, _, bv0=bv0, bv1=bv1):
            rows[r, pl.ds(0, 16)] += bv0
            rows[r, pl.ds(16, 16)] += bv1
            return 0

        lax.fori_loop(0, BLK, bbody, 0, unroll=8)

        pltpu.sync_copy(rows, out_hbm.at[pl.ds(b0, BLK), FC + f])


def kernel(x_continuous, x_categorical, w_nan, offsets, w_categorical,
           b_categorical, w_continuous, b_continuous):
    xcat_t = x_categorical.T
    xc_pad = jnp.pad(x_continuous, ((0, 0), (0, 16 - FC)))
    wnan_pad = jnp.pad(w_nan, (0, 16 - FC))
    offs2 = jnp.broadcast_to(offsets[:, None], (NF, 16))
    return _sc_embed(xc_pad, xcat_t, wnan_pad, offs2, w_categorical,
                     b_categorical, w_continuous, b_continuous)
